# baseline (speedup 1.0000x reference)
import functools

import jax
import jax.numpy as jnp
from jax import lax
from jax.experimental import pallas as pl
from jax.experimental.pallas import tpu as pltpu
from jax.experimental.pallas import tpu_sc as plsc

NPOL_K = 2
NANT_K = 128
NVIS_K = 8128
NFREQ_K = 256

S = 8
SUP = 16
NSUP = NVIS_K // SUP
NW = 32
FULL_W = NSUP - 15 * NW
KMAX = (NSUP + NW - 1) // NW
VROWS = NPOL_K * NPOL_K * NVIS_K * 4


def _body(v_hbm, j_hbm, p_hbm, q_hbm, out_hbm,
          p_vt, q_vt, jsh, jb1, jb2, vb, i1_v, i2_v, semv, semj, semo):
    cid = lax.axis_index("c")
    sid = lax.axis_index("s")
    wid = sid * 2 + cid

    pltpu.sync_copy(p_hbm.at[:, wid, :], p_vt)
    pltpu.sync_copy(q_hbm.at[:, wid, :], q_vt)

    nk = jnp.where(wid < FULL_W, KMAX, KMAX - 1)
    nsub = 2 * nk
    iot = lax.iota(jnp.int32, 16)

    def build_idx(k):
        kp = k & 1
        pvec = p_vt[k, :]
        qvec = q_vt[k, :]
        for pp in range(4):
            for fb in range(2):
                plsc.store_scatter(i1_v.at[kp, fb], [iot * 4 + pp],
                                   (pvec + pp * NANT_K) * 2 + fb)
                plsc.store_scatter(i2_v.at[kp, fb], [iot * 4 + pp],
                                   (qvec + pp * NANT_K) * 2 + fb)

    def vis0_of(n):
        return ((n >> 1) * NW + wid) * SUP + (n & 1) * S

    def issue_in_v(n, r):
        v0 = vis0_of(n)
        pltpu.async_copy(v_hbm.at[:, pl.ds(v0 * 4, 4 * S), :],
                         vb.at[r], semv.at[r])

    def wait_in_v(r):
        pltpu.make_async_copy(v_hbm.at[:, pl.ds(0, 4 * S), :],
                              vb.at[r], semv.at[r]).wait()

    def issue_j(n, fb):
        h = n & 1
        kp = (n >> 1) & 1
        sl = pl.ds(h * 4 * S, 4 * S)
        pltpu.async_copy(jsh.at[i1_v.at[kp, fb, sl]], jb1.at[fb], semj.at[fb])
        pltpu.async_copy(jsh.at[i2_v.at[kp, fb, sl]], jb2.at[fb], semj.at[fb])

    def wait_j(fb):
        pltpu.make_async_copy(j_hbm.at[pl.ds(0, 4 * S)], jb1.at[fb],
                              semj.at[fb]).wait()
        pltpu.make_async_copy(j_hbm.at[pl.ds(0, 4 * S)], jb2.at[fb],
                              semj.at[fb]).wait()

    def issue_out(n, r):
        v0 = vis0_of(n)
        pltpu.async_copy(vb.at[r],
                         out_hbm.at[:, pl.ds(v0 * 4, 4 * S), :], semo.at[r])

    def wait_out(r):
        pltpu.make_async_copy(vb.at[r], out_hbm.at[:, pl.ds(0, 4 * S), :],
                              semo.at[r]).wait()

    def compute_half(r, fb):
        for rs in range(3):
            @pl.when(r == rs)
            def _(rs=rs):
                _compute_half_static(rs, fb)

    def _compute_half_static(r, fb):
        rr = 2 * fb

        @plsc.parallel_loop(0, 8, unroll=2)
        def unit(u):
            band = u * 16
            for s_ in range(S):
                r0 = 4 * s_ + rr

                def ldm(pp, ri):
                    return vb[r, pp, r0 + ri, pl.ds(band, 16)]

                def ldj(jb, pp, ri):
                    return jb[fb, 4 * s_ + pp, ri, pl.ds(band, 16)]

                Mr = [[ldm(2 * i + j, 0) for j in range(2)] for i in range(2)]
                Mi = [[ldm(2 * i + j, 1) for j in range(2)] for i in range(2)]
                Ar = [[ldj(jb1, 2 * i + kk, 0) for kk in range(2)] for i in range(2)]
                Ai = [[ldj(jb1, 2 * i + kk, 1) for kk in range(2)] for i in range(2)]
                Br = [[ldj(jb2, 2 * j + kk, 0) for kk in range(2)] for j in range(2)]
                Bi = [[ldj(jb2, 2 * j + kk, 1) for kk in range(2)] for j in range(2)]

                Tr = [[Ar[i][0] * Mr[0][j] - Ai[i][0] * Mi[0][j]
                       + Ar[i][1] * Mr[1][j] - Ai[i][1] * Mi[1][j]
                       for j in range(2)] for i in range(2)]
                Ti = [[Ar[i][0] * Mi[0][j] + Ai[i][0] * Mr[0][j]
                       + Ar[i][1] * Mi[1][j] + Ai[i][1] * Mr[1][j]
                       for j in range(2)] for i in range(2)]

                for i in range(2):
                    for j in range(2):
                        orr = (Tr[i][0] * Br[j][0] + Ti[i][0] * Bi[j][0]
                               + Tr[i][1] * Br[j][1] + Ti[i][1] * Bi[j][1])
                        oii = (Ti[i][0] * Br[j][0] - Tr[i][0] * Bi[j][0]
                               + Ti[i][1] * Br[j][1] - Tr[i][1] * Bi[j][1])
                        vb[r, 2 * i + j, r0, pl.ds(band, 16)] = orr
                        vb[r, 2 * i + j, r0 + 1, pl.ds(band, 16)] = oii

    issue_in_v(jnp.int32(0), jnp.int32(0))
    pltpu.sync_copy(j_hbm.at[pl.ds(sid * 64, 64)], jsh.at[pl.ds(sid * 64, 64)])
    build_idx(jnp.int32(0))
    plsc.subcore_barrier()
    issue_j(jnp.int32(0), 0)
    issue_j(jnp.int32(0), 1)

    def sub_body(n, carry):
        r = lax.rem(n, 3)
        more = n + 1 < nsub
        wait_in_v(r)

        @pl.when(more)
        def _prefetch():
            nn = n + 1
            nr = lax.rem(nn, 3)

            @pl.when((nn & 1) == 0)
            def _():
                build_idx(nn >> 1)

            @pl.when(n >= 2)
            def _():
                wait_out(nr)

            issue_in_v(nn, nr)

        wait_j(0)
        compute_half(r, 0)

        @pl.when(more)
        def _g0():
            issue_j(n + 1, 0)

        wait_j(1)
        compute_half(r, 1)

        @pl.when(more)
        def _g1():
            issue_j(n + 1, 1)

        issue_out(n, r)
        return carry

    lax.fori_loop(0, nsub, sub_body, 0)

    for d in range(3):
        wait_out(lax.rem(nsub - 1 - d + 3, 3))


@jax.jit
def _jones_apply(v2, j3, p, q):
    mesh = plsc.VectorSubcoreMesh(core_axis_name="c", subcore_axis_name="s")
    f = functools.partial(
        pl.kernel,
        mesh=mesh,
        compiler_params=pltpu.CompilerParams(
            needs_layout_passes=False, use_tc_tiling_on_sc=False),
        out_type=jax.ShapeDtypeStruct((4, NVIS_K * 4, 128), jnp.float32),
        scratch_types=[
            pltpu.VMEM((KMAX, SUP), jnp.int32),
            pltpu.VMEM((KMAX, SUP), jnp.int32),
            pltpu.VMEM_SHARED((2 * 4 * NANT_K, 2, 128), jnp.float32),
            pltpu.VMEM((2, 4 * S, 2, 128), jnp.float32),
            pltpu.VMEM((2, 4 * S, 2, 128), jnp.float32),
            pltpu.VMEM((3, 4, 4 * S, 128), jnp.float32),
            pltpu.VMEM((2, 2, 4 * SUP), jnp.int32),
            pltpu.VMEM((2, 2, 4 * SUP), jnp.int32),
            pltpu.SemaphoreType.DMA((3,)),
            pltpu.SemaphoreType.DMA((2,)),
            pltpu.SemaphoreType.DMA((3,)),
        ],
    )(_body)
    return f(v2, j3, p, q)


def kernel(V_m, jones, vis2ants):
    pq = vis2ants.astype(jnp.int32)
    v2 = (V_m.reshape(NPOL_K, NPOL_K, NVIS_K, 2, 128, 2)
          .transpose(0, 1, 2, 3, 5, 4)
          .reshape(4, NVIS_K * 4, 128))
    j3 = (jones.reshape(NPOL_K, NPOL_K, NANT_K, 2, 128, 2)
          .transpose(0, 1, 2, 3, 5, 4)
          .reshape(2 * 4 * NANT_K, 2, 128))
    p = jnp.pad(pq[:, 0], (0, KMAX * NW * SUP - NVIS_K)).reshape(KMAX, NW, SUP)
    q = jnp.pad(pq[:, 1], (0, KMAX * NW * SUP - NVIS_K)).reshape(KMAX, NW, SUP)
    out = _jones_apply(v2, j3, p, q)
    return (out.reshape(NPOL_K, NPOL_K, NVIS_K, 2, 2, 128)
            .transpose(0, 1, 2, 3, 5, 4)
            .reshape(NPOL_K, NPOL_K, NVIS_K, NFREQ_K, 2))


# speedup vs baseline: 1.4500x; 1.4500x over previous
import functools

import jax
import jax.numpy as jnp
from jax import lax
from jax.experimental import pallas as pl
from jax.experimental.pallas import tpu as pltpu
from jax.experimental.pallas import tpu_sc as plsc

NPOL_K = 2
NANT_K = 128
NVIS_K = 8128
NFREQ_K = 256

S = 8
SUP = 16
NSUP = NVIS_K // SUP
NW = 32
FULL_W = NSUP - 15 * NW
KMAX = (NSUP + NW - 1) // NW
VROWS = NPOL_K * NPOL_K * NVIS_K * 4


def _body(v_hbm, j_hbm, p_hbm, q_hbm, out_hbm,
          p_vt, q_vt, jsh, jb1, jb2, vb, i1_v, i2_v, semv, semj, semo):
    cid = lax.axis_index("c")
    sid = lax.axis_index("s")
    wid = sid * 2 + cid

    pltpu.sync_copy(p_hbm.at[:, wid, :], p_vt)
    pltpu.sync_copy(q_hbm.at[:, wid, :], q_vt)

    nk = jnp.where(wid < FULL_W, KMAX, KMAX - 1)
    nsub = 2 * nk
    iot = lax.iota(jnp.int32, 16)

    def build_idx(k):
        kp = k & 1
        pvec = p_vt[k, :]
        qvec = q_vt[k, :]
        for pp in range(4):
            for fb in range(2):
                plsc.store_scatter(i1_v.at[kp, fb], [iot * 4 + pp],
                                   (pvec + pp * NANT_K) * 2 + fb)
                plsc.store_scatter(i2_v.at[kp, fb], [iot * 4 + pp],
                                   (qvec + pp * NANT_K) * 2 + fb)

    def vis0_of(n):
        return ((n >> 1) * NW + wid) * SUP + (n & 1) * S

    def issue_in_v(n, r):
        v0 = vis0_of(n)
        pltpu.async_copy(v_hbm.at[:, pl.ds(v0 * 4, 4 * S), :],
                         vb.at[r], semv.at[r])

    def wait_in_v(r):
        pltpu.make_async_copy(v_hbm.at[:, pl.ds(0, 4 * S), :],
                              vb.at[r], semv.at[r]).wait()

    def issue_j(n, fb):
        h = n & 1
        kp = (n >> 1) & 1
        sl = pl.ds(h * 4 * S, 4 * S)
        pltpu.async_copy(jsh.at[i1_v.at[kp, fb, sl]], jb1.at[fb], semj.at[fb])
        pltpu.async_copy(jsh.at[i2_v.at[kp, fb, sl]], jb2.at[fb], semj.at[fb])

    def wait_j(fb):
        pltpu.make_async_copy(j_hbm.at[pl.ds(0, 4 * S)], jb1.at[fb],
                              semj.at[fb]).wait()
        pltpu.make_async_copy(j_hbm.at[pl.ds(0, 4 * S)], jb2.at[fb],
                              semj.at[fb]).wait()

    def issue_out(n, r):
        v0 = vis0_of(n)
        pltpu.async_copy(vb.at[r],
                         out_hbm.at[:, pl.ds(v0 * 4, 4 * S), :], semo.at[r])

    def wait_out(r):
        pltpu.make_async_copy(vb.at[r], out_hbm.at[:, pl.ds(0, 4 * S), :],
                              semo.at[r]).wait()

    def compute_half(r, fb):
        for rs in range(3):
            @pl.when(r == rs)
            def _(rs=rs):
                _compute_half_static(rs, fb)

    def _compute_half_static(r, fb):
        rr = 2 * fb

        @plsc.parallel_loop(0, 8)
        def unit(u):
            band = u * 16
            for s_ in range(S):
                r0 = 4 * s_ + rr

                def ldm(pp, ri):
                    return vb[r, pp, r0 + ri, pl.ds(band, 16)]

                def ldj(jb, pp, ri):
                    return jb[fb, 4 * s_ + pp, ri, pl.ds(band, 16)]

                Mr = [[ldm(2 * i + j, 0) for j in range(2)] for i in range(2)]
                Mi = [[ldm(2 * i + j, 1) for j in range(2)] for i in range(2)]
                Ar = [[ldj(jb1, 2 * i + kk, 0) for kk in range(2)] for i in range(2)]
                Ai = [[ldj(jb1, 2 * i + kk, 1) for kk in range(2)] for i in range(2)]
                Br = [[ldj(jb2, 2 * j + kk, 0) for kk in range(2)] for j in range(2)]
                Bi = [[ldj(jb2, 2 * j + kk, 1) for kk in range(2)] for j in range(2)]

                Tr = [[Ar[i][0] * Mr[0][j] - Ai[i][0] * Mi[0][j]
                       + Ar[i][1] * Mr[1][j] - Ai[i][1] * Mi[1][j]
                       for j in range(2)] for i in range(2)]
                Ti = [[Ar[i][0] * Mi[0][j] + Ai[i][0] * Mr[0][j]
                       + Ar[i][1] * Mi[1][j] + Ai[i][1] * Mr[1][j]
                       for j in range(2)] for i in range(2)]

                for i in range(2):
                    for j in range(2):
                        orr = (Tr[i][0] * Br[j][0] + Ti[i][0] * Bi[j][0]
                               + Tr[i][1] * Br[j][1] + Ti[i][1] * Bi[j][1])
                        oii = (Ti[i][0] * Br[j][0] - Tr[i][0] * Bi[j][0]
                               + Ti[i][1] * Br[j][1] - Tr[i][1] * Bi[j][1])
                        vb[r, 2 * i + j, r0, pl.ds(band, 16)] = orr
                        vb[r, 2 * i + j, r0 + 1, pl.ds(band, 16)] = oii

    issue_in_v(jnp.int32(0), jnp.int32(0))
    pltpu.sync_copy(j_hbm.at[pl.ds(sid * 64, 64)], jsh.at[pl.ds(sid * 64, 64)])
    build_idx(jnp.int32(0))
    plsc.subcore_barrier()
    issue_j(jnp.int32(0), 0)
    issue_j(jnp.int32(0), 1)

    def sub_body(n, carry):
        r = lax.rem(n, 3)
        more = n + 1 < nsub
        wait_in_v(r)

        @pl.when(more)
        def _prefetch():
            nn = n + 1
            nr = lax.rem(nn, 3)

            @pl.when((nn & 1) == 0)
            def _():
                build_idx(nn >> 1)

            @pl.when(n >= 2)
            def _():
                wait_out(nr)

            issue_in_v(nn, nr)

        wait_j(0)
        compute_half(r, 0)

        @pl.when(more)
        def _g0():
            issue_j(n + 1, 0)

        wait_j(1)
        compute_half(r, 1)

        @pl.when(more)
        def _g1():
            issue_j(n + 1, 1)

        issue_out(n, r)
        return carry

    lax.fori_loop(0, nsub, sub_body, 0)

    for d in range(3):
        wait_out(lax.rem(nsub - 1 - d + 3, 3))


@jax.jit
def _jones_apply(v2, j3, p, q):
    mesh = plsc.VectorSubcoreMesh(core_axis_name="c", subcore_axis_name="s")
    f = functools.partial(
        pl.kernel,
        mesh=mesh,
        compiler_params=pltpu.CompilerParams(
            needs_layout_passes=False, use_tc_tiling_on_sc=False),
        out_type=jax.ShapeDtypeStruct((4, NVIS_K * 4, 128), jnp.float32),
        scratch_types=[
            pltpu.VMEM((KMAX, SUP), jnp.int32),
            pltpu.VMEM((KMAX, SUP), jnp.int32),
            pltpu.VMEM_SHARED((2 * 4 * NANT_K, 2, 128), jnp.float32),
            pltpu.VMEM((2, 4 * S, 2, 128), jnp.float32),
            pltpu.VMEM((2, 4 * S, 2, 128), jnp.float32),
            pltpu.VMEM((3, 4, 4 * S, 128), jnp.float32),
            pltpu.VMEM((2, 2, 4 * SUP), jnp.int32),
            pltpu.VMEM((2, 2, 4 * SUP), jnp.int32),
            pltpu.SemaphoreType.DMA((3,)),
            pltpu.SemaphoreType.DMA((2,)),
            pltpu.SemaphoreType.DMA((3,)),
        ],
    )(_body)
    return f(v2, j3, p, q)


def kernel(V_m, jones, vis2ants):
    pq = vis2ants.astype(jnp.int32)
    v2 = (V_m.reshape(NPOL_K, NPOL_K, NVIS_K, 2, 128, 2)
          .transpose(0, 1, 2, 3, 5, 4)
          .reshape(4, NVIS_K * 4, 128))
    j3 = (jones.reshape(NPOL_K, NPOL_K, NANT_K, 2, 128, 2)
          .transpose(0, 1, 2, 3, 5, 4)
          .reshape(2 * 4 * NANT_K, 2, 128))
    p = jnp.pad(pq[:, 0], (0, KMAX * NW * SUP - NVIS_K)).reshape(KMAX, NW, SUP)
    q = jnp.pad(pq[:, 1], (0, KMAX * NW * SUP - NVIS_K)).reshape(KMAX, NW, SUP)
    out = _jones_apply(v2, j3, p, q)
    return (out.reshape(NPOL_K, NPOL_K, NVIS_K, 2, 2, 128)
            .transpose(0, 1, 2, 3, 5, 4)
            .reshape(NPOL_K, NPOL_K, NVIS_K, NFREQ_K, 2))


# speedup vs baseline: 1.4800x; 1.0207x over previous
import functools

import jax
import jax.numpy as jnp
from jax import lax
from jax.experimental import pallas as pl
from jax.experimental.pallas import tpu as pltpu
from jax.experimental.pallas import tpu_sc as plsc

NPOL_K = 2
NANT_K = 128
NVIS_K = 8128
NFREQ_K = 256

S = 8
SUP = 16
NSUP = NVIS_K // SUP
NW = 32
FULL_W = NSUP - 15 * NW
KMAX = (NSUP + NW - 1) // NW
VROWS = NPOL_K * NPOL_K * NVIS_K * 4


def _body(v_hbm, j_hbm, p_hbm, q_hbm, out_hbm,
          p_vt, q_vt, jsh, jball, vb, iall, semv, semj, semo):
    cid = lax.axis_index("c")
    sid = lax.axis_index("s")
    wid = sid * 2 + cid

    pltpu.sync_copy(p_hbm.at[:, wid, :], p_vt)
    pltpu.sync_copy(q_hbm.at[:, wid, :], q_vt)

    nk = jnp.where(wid < FULL_W, KMAX, KMAX - 1)
    nsub = 2 * nk
    iot = lax.iota(jnp.int32, 16)

    def build_idx(k):
        kp = k & 1
        pvec = p_vt[k, :]
        qvec = q_vt[k, :]
        pos = iot * 4 + jnp.where(iot >= 8, 32, 0)
        for pp in range(4):
            for fb in range(2):
                plsc.store_scatter(iall.at[kp, fb], [pos + pp],
                                   (pvec + pp * NANT_K) * 2 + fb)
                plsc.store_scatter(iall.at[kp, fb], [pos + pp + 32],
                                   (qvec + pp * NANT_K) * 2 + fb)

    def vis0_of(n):
        return ((n >> 1) * NW + wid) * SUP + (n & 1) * S

    def issue_in_v(n, r):
        v0 = vis0_of(n)
        pltpu.async_copy(v_hbm.at[:, pl.ds(v0 * 4, 4 * S), :],
                         vb.at[r], semv.at[r])

    def wait_in_v(r):
        pltpu.make_async_copy(v_hbm.at[:, pl.ds(0, 4 * S), :],
                              vb.at[r], semv.at[r]).wait()

    def issue_j(n, fb):
        h = n & 1
        kp = (n >> 1) & 1
        sl = pl.ds(h * 8 * S, 8 * S)
        pltpu.async_copy(jsh.at[iall.at[kp, fb, sl]], jball.at[fb], semj.at[fb])

    def wait_j(fb):
        pltpu.make_async_copy(j_hbm.at[pl.ds(0, 8 * S)], jball.at[fb],
                              semj.at[fb]).wait()

    def issue_out(n, r):
        v0 = vis0_of(n)
        pltpu.async_copy(vb.at[r],
                         out_hbm.at[:, pl.ds(v0 * 4, 4 * S), :], semo.at[r])

    def wait_out(r):
        pltpu.make_async_copy(vb.at[r], out_hbm.at[:, pl.ds(0, 4 * S), :],
                              semo.at[r]).wait()

    def compute_half(r, fb):
        for rs in range(3):
            @pl.when(r == rs)
            def _(rs=rs):
                _compute_half_static(rs, fb)

    def _compute_half_static(r, fb):
        rr = 2 * fb

        @plsc.parallel_loop(0, 8)
        def unit(u):
            band = u * 16
            for s_ in range(S):
                r0 = 4 * s_ + rr

                def ldm(pp, ri):
                    return vb[r, pp, r0 + ri, pl.ds(band, 16)]

                def ldj(off, pp, ri):
                    return jball[fb, off + 4 * s_ + pp, ri, pl.ds(band, 16)]

                Mr = [[ldm(2 * i + j, 0) for j in range(2)] for i in range(2)]
                Mi = [[ldm(2 * i + j, 1) for j in range(2)] for i in range(2)]
                Ar = [[ldj(0, 2 * i + kk, 0) for kk in range(2)] for i in range(2)]
                Ai = [[ldj(0, 2 * i + kk, 1) for kk in range(2)] for i in range(2)]
                Br = [[ldj(32, 2 * j + kk, 0) for kk in range(2)] for j in range(2)]
                Bi = [[ldj(32, 2 * j + kk, 1) for kk in range(2)] for j in range(2)]

                Tr = [[Ar[i][0] * Mr[0][j] - Ai[i][0] * Mi[0][j]
                       + Ar[i][1] * Mr[1][j] - Ai[i][1] * Mi[1][j]
                       for j in range(2)] for i in range(2)]
                Ti = [[Ar[i][0] * Mi[0][j] + Ai[i][0] * Mr[0][j]
                       + Ar[i][1] * Mi[1][j] + Ai[i][1] * Mr[1][j]
                       for j in range(2)] for i in range(2)]

                for i in range(2):
                    for j in range(2):
                        orr = (Tr[i][0] * Br[j][0] + Ti[i][0] * Bi[j][0]
                               + Tr[i][1] * Br[j][1] + Ti[i][1] * Bi[j][1])
                        oii = (Ti[i][0] * Br[j][0] - Tr[i][0] * Bi[j][0]
                               + Ti[i][1] * Br[j][1] - Tr[i][1] * Bi[j][1])
                        vb[r, 2 * i + j, r0, pl.ds(band, 16)] = orr
                        vb[r, 2 * i + j, r0 + 1, pl.ds(band, 16)] = oii

    issue_in_v(jnp.int32(0), jnp.int32(0))
    pltpu.sync_copy(j_hbm.at[pl.ds(sid * 64, 64)], jsh.at[pl.ds(sid * 64, 64)])
    build_idx(jnp.int32(0))
    plsc.subcore_barrier()
    issue_j(jnp.int32(0), 0)
    issue_j(jnp.int32(0), 1)

    def sub_body(n, carry):
        r = lax.rem(n, 3)
        more = n + 1 < nsub
        wait_in_v(r)

        @pl.when(more)
        def _prefetch():
            nn = n + 1
            nr = lax.rem(nn, 3)

            @pl.when((nn & 1) == 0)
            def _():
                build_idx(nn >> 1)

            @pl.when(n >= 2)
            def _():
                wait_out(nr)

            issue_in_v(nn, nr)

        wait_j(0)
        compute_half(r, 0)

        @pl.when(more)
        def _g0():
            issue_j(n + 1, 0)

        wait_j(1)
        compute_half(r, 1)

        @pl.when(more)
        def _g1():
            issue_j(n + 1, 1)

        issue_out(n, r)
        return carry

    lax.fori_loop(0, nsub, sub_body, 0)

    for d in range(3):
        wait_out(lax.rem(nsub - 1 - d + 3, 3))


@jax.jit
def _jones_apply(v2, j3, p, q):
    mesh = plsc.VectorSubcoreMesh(core_axis_name="c", subcore_axis_name="s")
    f = functools.partial(
        pl.kernel,
        mesh=mesh,
        compiler_params=pltpu.CompilerParams(
            needs_layout_passes=False, use_tc_tiling_on_sc=False),
        out_type=jax.ShapeDtypeStruct((4, NVIS_K * 4, 128), jnp.float32),
        scratch_types=[
            pltpu.VMEM((KMAX, SUP), jnp.int32),
            pltpu.VMEM((KMAX, SUP), jnp.int32),
            pltpu.VMEM_SHARED((2 * 4 * NANT_K, 2, 128), jnp.float32),
            pltpu.VMEM((2, 8 * S, 2, 128), jnp.float32),
            pltpu.VMEM((3, 4, 4 * S, 128), jnp.float32),
            pltpu.VMEM((2, 2, 8 * SUP), jnp.int32),
            pltpu.SemaphoreType.DMA((3,)),
            pltpu.SemaphoreType.DMA((2,)),
            pltpu.SemaphoreType.DMA((3,)),
        ],
    )(_body)
    return f(v2, j3, p, q)


def kernel(V_m, jones, vis2ants):
    pq = vis2ants.astype(jnp.int32)
    v2 = (V_m.reshape(NPOL_K, NPOL_K, NVIS_K, 2, 128, 2)
          .transpose(0, 1, 2, 3, 5, 4)
          .reshape(4, NVIS_K * 4, 128))
    j3 = (jones.reshape(NPOL_K, NPOL_K, NANT_K, 2, 128, 2)
          .transpose(0, 1, 2, 3, 5, 4)
          .reshape(2 * 4 * NANT_K, 2, 128))
    p = jnp.pad(pq[:, 0], (0, KMAX * NW * SUP - NVIS_K)).reshape(KMAX, NW, SUP)
    q = jnp.pad(pq[:, 1], (0, KMAX * NW * SUP - NVIS_K)).reshape(KMAX, NW, SUP)
    out = _jones_apply(v2, j3, p, q)
    return (out.reshape(NPOL_K, NPOL_K, NVIS_K, 2, 2, 128)
            .transpose(0, 1, 2, 3, 5, 4)
            .reshape(NPOL_K, NPOL_K, NVIS_K, NFREQ_K, 2))
